# Initial kernel scaffold; baseline (speedup 1.0000x reference)
import jax
import jax.numpy as jnp
from jax.experimental import pallas as pl


def kernel(h, coord, edge_index, coord_diff, edge_attr, node_mask, edge_mask, W1, b1, W2, b2, W3):
    raise NotImplementedError("write your pallas kernel here")


# speedup vs baseline: 2.2489x; 2.2489x over previous
import functools

import jax
import jax.numpy as jnp
from jax import lax
from jax.experimental import pallas as pl
from jax.experimental.pallas import tpu as pltpu
from jax.experimental.pallas import tpu_sc as plsc

N_NODES = 10000
N_PAD = 10240
N_EDGES = 320000
H = 128

NC = 2
NS = 16
NW = NC * NS
CHUNK = 128
NCHUNKS = N_EDGES // CHUNK
BASE_CH = NCHUNKS // NW
EXTRA = NCHUNKS - BASE_CH * NW
RPT = N_PAD // NS


def _tables_body(h_ref, wr_ref, wc_ref, pr_ref, pc_ref):
    hv = h_ref[...]
    pr_ref[...] = jnp.dot(hv, wr_ref[...], preferred_element_type=jnp.float32)
    pc_ref[...] = jnp.dot(hv, wc_ref[...], preferred_element_type=jnp.float32)


def _make_tables(h, w_rt, w_ct):
    return pl.pallas_call(
        _tables_body,
        out_shape=(
            jax.ShapeDtypeStruct((N_NODES, H), jnp.float32),
            jax.ShapeDtypeStruct((N_NODES, H), jnp.float32),
        ),
    )(h, w_rt, w_ct)


def _gather_kernel(pr_hbm, pc_hbm, row_hbm, col_hbm, r_out, c_out,
                   idxr, idxc, rbuf, cbuf, sem1, sem2):
    wid = lax.axis_index("s") * NC + lax.axis_index("c")
    nch = BASE_CH + jnp.where(wid < EXTRA, 1, 0)

    def chunk(i, _):
        off = (wid + i * NW) * CHUNK
        ds = pl.ds(off, CHUNK)
        pltpu.sync_copy(row_hbm.at[ds], idxr)
        pltpu.sync_copy(col_hbm.at[ds], idxc)
        gr = pltpu.async_copy(pr_hbm.at[idxr], rbuf, sem1)
        gc = pltpu.async_copy(pc_hbm.at[idxc], cbuf, sem2)
        gr.wait()
        gc.wait()
        wr = pltpu.async_copy(rbuf, r_out.at[ds], sem1)
        wc = pltpu.async_copy(cbuf, c_out.at[ds], sem2)
        wr.wait()
        wc.wait()
        return ()

    lax.fori_loop(0, nch, chunk, ())


def _gather(pre_row, pre_col, row_idx, col_idx):
    k = functools.partial(
        pl.kernel,
        out_type=(
            jax.ShapeDtypeStruct((N_EDGES, H), jnp.float32),
            jax.ShapeDtypeStruct((N_EDGES, H), jnp.float32),
        ),
        mesh=plsc.VectorSubcoreMesh(core_axis_name="c", subcore_axis_name="s"),
        scratch_types=[
            pltpu.VMEM((CHUNK,), jnp.int32),
            pltpu.VMEM((CHUNK,), jnp.int32),
            pltpu.VMEM((CHUNK, H), jnp.float32),
            pltpu.VMEM((CHUNK, H), jnp.float32),
            pltpu.SemaphoreType.DMA,
            pltpu.SemaphoreType.DMA,
        ],
    )(_gather_kernel)
    return k(pre_row, pre_col, row_idx, col_idx)


def _mlp_body(r_ref, c_ref, ea_ref, cd3_ref, em_ref, w1ct_ref, b1_ref,
              w2t_ref, b2_ref, w3_ref, tx_ref, ty_ref, tz_ref):
    e1 = jnp.dot(ea_ref[...], w1ct_ref[...], preferred_element_type=jnp.float32)
    x1 = jax.nn.silu(r_ref[...] + c_ref[...] + e1 + b1_ref[...])
    x2 = jax.nn.silu(
        jnp.dot(x1, w2t_ref[...], preferred_element_type=jnp.float32)
        + b2_ref[...])
    m = jnp.sum(x2 * w3_ref[...], axis=1, keepdims=True)
    mscale = m * em_ref[...] * (1.0 / 100.0)
    trans = cd3_ref[...] * mscale
    tx_ref[...] = trans[:, 0:1]
    ty_ref[...] = trans[:, 1:2]
    tz_ref[...] = trans[:, 2:3]


def _mlp(r, c, edge_attr, cd3, edge_mask, w1ct, b1, w2t, b2, w3):
    EB = 2000
    grid = N_EDGES // EB
    return pl.pallas_call(
        _mlp_body,
        grid=(grid,),
        in_specs=[
            pl.BlockSpec((EB, H), lambda i: (i, 0)),
            pl.BlockSpec((EB, H), lambda i: (i, 0)),
            pl.BlockSpec((EB, 4), lambda i: (i, 0)),
            pl.BlockSpec((EB, 3), lambda i: (i, 0)),
            pl.BlockSpec((EB, 1), lambda i: (i, 0)),
            pl.BlockSpec((4, H), lambda i: (0, 0)),
            pl.BlockSpec((1, H), lambda i: (0, 0)),
            pl.BlockSpec((H, H), lambda i: (0, 0)),
            pl.BlockSpec((1, H), lambda i: (0, 0)),
            pl.BlockSpec((1, H), lambda i: (0, 0)),
        ],
        out_specs=[pl.BlockSpec((EB, 1), lambda i: (i, 0))] * 3,
        out_shape=tuple(jax.ShapeDtypeStruct((N_EDGES, 1), jnp.float32)
                        for _ in range(3)),
    )(r, c, edge_attr, cd3, edge_mask, w1ct, b1, w2t, b2, w3)


def _scatter_kernel(tx, ty, tz, row_hbm, zero_hbm, px, py, pz,
                    accx, accy, accz, bx, by, bz, ibuf):
    wid = lax.axis_index("s") * NC + lax.axis_index("c")
    nch = BASE_CH + jnp.where(wid < EXTRA, 1, 0)

    pltpu.sync_copy(zero_hbm, accx)
    pltpu.sync_copy(zero_hbm, accy)
    pltpu.sync_copy(zero_hbm, accz)
    iota = lax.iota(jnp.int32, 16)

    def chunk(i, _):
        off = (wid + i * NW) * CHUNK
        ds = pl.ds(off, CHUNK)
        pltpu.sync_copy(row_hbm.at[ds], ibuf)
        pltpu.sync_copy(tx.at[ds], bx)
        pltpu.sync_copy(ty.at[ds], by)
        pltpu.sync_copy(tz.at[ds], bz)

        def vreg(j, _):
            sl = pl.ds(j * 16, 16)
            ivec = ibuf[sl]
            xv = bx[sl]
            yv = by[sl]
            zv = bz[sl]
            for l in range(16):
                i = ivec[l]
                b = i & ~15
                m = iota == (i - b)
                w = pl.ds(b, 16)
                plsc.addupdate(accx.at[w], jnp.where(m, xv[l], 0.0))
                plsc.addupdate(accy.at[w], jnp.where(m, yv[l], 0.0))
                plsc.addupdate(accz.at[w], jnp.where(m, zv[l], 0.0))
            return ()

        lax.fori_loop(0, CHUNK // 16, vreg, ())
        return ()

    lax.fori_loop(0, nch, chunk, ())

    out_ds = pl.ds(wid * N_PAD, N_PAD)
    pltpu.sync_copy(accx, px.at[out_ds])
    pltpu.sync_copy(accy, py.at[out_ds])
    pltpu.sync_copy(accz, pz.at[out_ds])


def _scatter(tx, ty, tz, row_idx, zero_rows):
    k = functools.partial(
        pl.kernel,
        out_type=tuple(jax.ShapeDtypeStruct((NW * N_PAD,), jnp.float32)
                       for _ in range(3)),
        mesh=plsc.VectorSubcoreMesh(core_axis_name="c", subcore_axis_name="s"),
        scratch_types=[
            pltpu.VMEM((N_PAD,), jnp.float32),
            pltpu.VMEM((N_PAD,), jnp.float32),
            pltpu.VMEM((N_PAD,), jnp.float32),
            pltpu.VMEM((CHUNK,), jnp.float32),
            pltpu.VMEM((CHUNK,), jnp.float32),
            pltpu.VMEM((CHUNK,), jnp.float32),
            pltpu.VMEM((CHUNK,), jnp.int32),
        ],
    )(_scatter_kernel)
    return k(tx, ty, tz, row_idx, zero_rows)


def _final_body(coordt_ref, nmt_ref, px_ref, py_ref, pz_ref, out_ref):
    sx = jnp.sum(px_ref[...], axis=0)[:N_NODES]
    sy = jnp.sum(py_ref[...], axis=0)[:N_NODES]
    sz = jnp.sum(pz_ref[...], axis=0)[:N_NODES]
    agg = jnp.concatenate([sx[None, :], sy[None, :], sz[None, :]], axis=0)
    out_ref[...] = (coordt_ref[...] + agg) * nmt_ref[...]


def _finalize(coordt, nmt, px, py, pz):
    return pl.pallas_call(
        _final_body,
        out_shape=jax.ShapeDtypeStruct((3, N_NODES), jnp.float32),
    )(coordt, nmt, px, py, pz)


def kernel(h, coord, edge_index, coord_diff, edge_attr, node_mask, edge_mask,
           W1, b1, W2, b2, W3):
    row = edge_index[0].astype(jnp.int32)
    col = edge_index[1].astype(jnp.int32)
    w_rt = W1[:, :H].T
    w_ct = W1[:, H:2 * H].T
    w1ct = W1[:, 2 * H:].T
    zero_rows = jnp.zeros((N_PAD,), jnp.float32)

    pre_row, pre_col = _make_tables(h, w_rt, w_ct)
    r, c = _gather(pre_row, pre_col, row, col)
    tx, ty, tz = _mlp(r, c, edge_attr, coord_diff, edge_mask, w1ct,
                      b1.reshape(1, H), W2.T, b2.reshape(1, H), W3.reshape(1, H))
    px, py, pz = _scatter(tx.reshape(N_EDGES), ty.reshape(N_EDGES),
                          tz.reshape(N_EDGES), row, zero_rows)
    outt = _finalize(coord.T, node_mask.T,
                     px.reshape(NW, N_PAD), py.reshape(NW, N_PAD),
                     pz.reshape(NW, N_PAD))
    return outt.T
